# baseline (speedup 1.0000x reference)
import functools

import jax
import jax.numpy as jnp
from jax.experimental import pallas as pl
from jax.experimental.pallas import tpu as pltpu

_TOPK = 64
_POS_INF_BITS = 0x7F800000


_DNUMS_RHS_T = (((1,), (1,)), ((), ()))


def _enc_body(x_ref, w_ref, b_ref, out_ref):
    z = jax.lax.dot_general(x_ref[...], w_ref[...], _DNUMS_RHS_T,
                            preferred_element_type=jnp.float32)
    out_ref[...] = jnp.maximum(z + b_ref[...], 0.0)


def _gate_body(lat_ref, thr_ref):
    z = lat_ref[...]
    bits = jax.lax.bitcast_convert_type(z, jnp.int32)
    r = z.shape[0]
    lo0 = jnp.zeros((r, 1), jnp.int32)
    hi0 = jnp.full((r, 1), _POS_INF_BITS, jnp.int32)

    def body(_, carry):
        lo, hi = carry
        mid = lo + jax.lax.shift_right_logical(hi - lo, 1)
        cnt = jnp.sum((bits >= mid).astype(jnp.int32), axis=1, keepdims=True)
        ge = cnt >= _TOPK
        return jnp.where(ge, mid, lo), jnp.where(ge, hi, mid)

    lo, _ = jax.lax.fori_loop(0, 31, body, (lo0, hi0))
    thr_ref[...] = jax.lax.bitcast_convert_type(lo, jnp.float32)


def _dec_body(lat_ref, thr_ref, w_ref, b_ref, out_ref, acc_ref, *, nk):
    k = pl.program_id(1)

    @pl.when(k == 0)
    def _init():
        acc_ref[...] = jnp.zeros_like(acc_ref)

    z = lat_ref[...]
    g = jnp.where(z >= thr_ref[...], z, 0.0).astype(jnp.bfloat16)
    acc_ref[...] += jax.lax.dot_general(g, w_ref[...], _DNUMS_RHS_T,
                                        preferred_element_type=jnp.float32)

    @pl.when(k == nk - 1)
    def _fin():
        out_ref[...] = acc_ref[...] + b_ref[...]


def kernel(x, W_enc, b_enc, W_dec, b_dec):
    B, L, D = x.shape
    Dl = W_enc.shape[0]
    N = B * L
    xf = x.reshape(N, D)
    b_enc2 = b_enc.reshape(1, Dl)
    b_dec2 = b_dec.reshape(1, D)

    r1 = min(1024, N)
    dlb = min(1024, Dl)
    latent = pl.pallas_call(
        _enc_body,
        grid=(N // r1, Dl // dlb),
        in_specs=[
            pl.BlockSpec((r1, D), lambda i, j: (i, 0)),
            pl.BlockSpec((dlb, D), lambda i, j: (j, 0)),
            pl.BlockSpec((1, dlb), lambda i, j: (0, j)),
        ],
        out_specs=pl.BlockSpec((r1, dlb), lambda i, j: (i, j)),
        out_shape=jax.ShapeDtypeStruct((N, Dl), jnp.float32),
    )(xf, W_enc, b_enc2)

    r2 = min(512, N)
    thr = pl.pallas_call(
        _gate_body,
        grid=(N // r2,),
        in_specs=[pl.BlockSpec((r2, Dl), lambda i: (i, 0))],
        out_specs=pl.BlockSpec((r2, 1), lambda i: (i, 0)),
        out_shape=jax.ShapeDtypeStruct((N, 1), jnp.float32),
    )(latent)

    r3 = min(1024, N)
    kb = min(1024, Dl)
    nk = Dl // kb
    mod = pl.pallas_call(
        functools.partial(_dec_body, nk=nk),
        grid=(N // r3, nk),
        in_specs=[
            pl.BlockSpec((r3, kb), lambda i, k: (i, k)),
            pl.BlockSpec((r3, 1), lambda i, k: (i, 0)),
            pl.BlockSpec((D, kb), lambda i, k: (0, k)),
            pl.BlockSpec((1, D), lambda i, k: (0, 0)),
        ],
        out_specs=pl.BlockSpec((r3, D), lambda i, k: (i, 0)),
        out_shape=jax.ShapeDtypeStruct((N, D), jnp.float32),
        scratch_shapes=[pltpu.VMEM((r3, D), jnp.float32)],
    )(latent, thr, W_dec.astype(jnp.bfloat16), b_dec2)

    return mod.reshape(B, L, D), latent


# speedup vs baseline: 1.1609x; 1.1609x over previous
import functools

import jax
import jax.numpy as jnp
from jax.experimental import pallas as pl
from jax.experimental.pallas import tpu as pltpu

_TOPK = 64
_POS_INF_BITS = 0x7F800000


_DNUMS_RHS_T = (((1,), (1,)), ((), ()))


def _enc_body(x_ref, w_ref, b_ref, out_ref):
    z = jax.lax.dot_general(x_ref[...], w_ref[...], _DNUMS_RHS_T,
                            preferred_element_type=jnp.float32)
    out_ref[...] = jnp.maximum(z + b_ref[...], 0.0)


_COARSE_ITERS = 18


def _gate_body(lat_ref, thr_ref):
    z = lat_ref[...]
    bits = jax.lax.bitcast_convert_type(z, jnp.int32)
    r = z.shape[0]
    lo0 = jnp.zeros((r, 1), jnp.int32)
    hi0 = jnp.full((r, 1), _POS_INF_BITS, jnp.int32)
    chi0 = jnp.zeros((r, 1), jnp.int32)

    def coarse(_, carry):
        lo, hi, cnt_hi = carry
        mid = lo + jax.lax.shift_right_logical(hi - lo, 1)
        cnt = jnp.sum((bits >= mid).astype(jnp.int32), axis=1, keepdims=True)
        ge = cnt >= _TOPK
        return (jnp.where(ge, mid, lo), jnp.where(ge, hi, mid),
                jnp.where(ge, cnt_hi, cnt))

    lo, hi, cnt_hi = jax.lax.fori_loop(0, _COARSE_ITERS, coarse,
                                       (lo0, hi0, chi0))

    t0 = jax.lax.bitcast_convert_type(hi, jnp.float32)

    def fine_cond(carry):
        _, cnt = carry
        return jnp.any(cnt < _TOPK)

    def fine_body(carry):
        t, cnt = carry
        nxt = jnp.max(jnp.where(z < t, z, -jnp.inf), axis=1, keepdims=True)
        t2 = jnp.where(cnt < _TOPK, nxt, t)
        cnt2 = jnp.sum((z >= t2).astype(jnp.int32), axis=1, keepdims=True)
        return t2, cnt2

    t, _ = jax.lax.while_loop(fine_cond, fine_body, (t0, cnt_hi))
    thr_ref[...] = t


def _dec_body(lat_ref, thr_ref, w_ref, b_ref, out_ref, acc_ref, *, nk):
    k = pl.program_id(1)

    @pl.when(k == 0)
    def _init():
        acc_ref[...] = jnp.zeros_like(acc_ref)

    z = lat_ref[...]
    g = jnp.where(z >= thr_ref[...], z, 0.0)
    acc_ref[...] += jax.lax.dot_general(g, w_ref[...], _DNUMS_RHS_T,
                                        preferred_element_type=jnp.float32)

    @pl.when(k == nk - 1)
    def _fin():
        out_ref[...] = acc_ref[...] + b_ref[...]


def kernel(x, W_enc, b_enc, W_dec, b_dec):
    B, L, D = x.shape
    Dl = W_enc.shape[0]
    N = B * L
    xf = x.reshape(N, D)
    b_enc2 = b_enc.reshape(1, Dl)
    b_dec2 = b_dec.reshape(1, D)

    r1 = min(1024, N)
    dlb = min(1024, Dl)
    latent = pl.pallas_call(
        _enc_body,
        grid=(N // r1, Dl // dlb),
        in_specs=[
            pl.BlockSpec((r1, D), lambda i, j: (i, 0)),
            pl.BlockSpec((dlb, D), lambda i, j: (j, 0)),
            pl.BlockSpec((1, dlb), lambda i, j: (0, j)),
        ],
        out_specs=pl.BlockSpec((r1, dlb), lambda i, j: (i, j)),
        out_shape=jax.ShapeDtypeStruct((N, Dl), jnp.float32),
    )(xf, W_enc, b_enc2)

    r2 = min(512, N)
    thr = pl.pallas_call(
        _gate_body,
        grid=(N // r2,),
        in_specs=[pl.BlockSpec((r2, Dl), lambda i: (i, 0))],
        out_specs=pl.BlockSpec((r2, 1), lambda i: (i, 0)),
        out_shape=jax.ShapeDtypeStruct((N, 1), jnp.float32),
    )(latent)

    r3 = min(1024, N)
    kb = min(1024, Dl)
    nk = Dl // kb
    mod = pl.pallas_call(
        functools.partial(_dec_body, nk=nk),
        grid=(N // r3, nk),
        in_specs=[
            pl.BlockSpec((r3, kb), lambda i, k: (i, k)),
            pl.BlockSpec((r3, 1), lambda i, k: (i, 0)),
            pl.BlockSpec((D, kb), lambda i, k: (0, k)),
            pl.BlockSpec((1, D), lambda i, k: (0, 0)),
        ],
        out_specs=pl.BlockSpec((r3, D), lambda i, k: (i, 0)),
        out_shape=jax.ShapeDtypeStruct((N, D), jnp.float32),
        scratch_shapes=[pltpu.VMEM((r3, D), jnp.float32)],
    )(latent, thr, W_dec, b_dec2)

    return mod.reshape(B, L, D), latent
